# baseline (speedup 1.0000x reference)
import functools
import math

import jax
import jax.numpy as jnp
from jax import lax
from jax.experimental import pallas as pl
from jax.experimental.pallas import tpu as pltpu
from jax.experimental.pallas import tpu_sc as plsc

D_MODEL = 1024
N_HEADS = 16
N_MODULES = 8
TOPK_E = 2
CAP = 256
T_TOK = 2048
HD = D_MODEL // N_HEADS
ROPE_BASE_C = 10000.0

_NC = 2
_NS = 16
_NW = _NC * _NS


def _sc_embed_gather(table, idx, zrows, B, D):
    bpw = B // _NW
    mesh = plsc.VectorSubcoreMesh(core_axis_name="c", subcore_axis_name="s")

    @functools.partial(
        pl.kernel, mesh=mesh,
        out_type=jax.ShapeDtypeStruct((B + 8, D), jnp.float32),
        scratch_types=[
            pltpu.VMEM((bpw // 2,), jnp.int32),
            pltpu.VMEM((bpw // 2,), jnp.int32),
            pltpu.VMEM((bpw // 2, D), jnp.float32),
            pltpu.VMEM((bpw // 2, D), jnp.float32),
            pltpu.VMEM((8, D), jnp.float32),
            pltpu.SemaphoreType.DMA,
            pltpu.SemaphoreType.DMA,
            pltpu.SemaphoreType.DMA,
            pltpu.SemaphoreType.DMA,
        ],
    )
    def k(table_hbm, idx_hbm, z_hbm, out_hbm, i0, i1, r0, r1, z_v,
          sg0, sg1, sw0, sw1):
        wid = lax.axis_index("s") * _NC + lax.axis_index("c")
        base = wid * bpw
        half = bpw // 2
        pltpu.sync_copy(idx_hbm.at[pl.ds(base, half)], i0)
        g0 = pltpu.async_copy(table_hbm.at[i0], r0, sg0)
        pltpu.sync_copy(idx_hbm.at[pl.ds(base + half, half)], i1)
        g1 = pltpu.async_copy(table_hbm.at[i1], r1, sg1)
        g0.wait()
        w0 = pltpu.async_copy(r0, out_hbm.at[pl.ds(base, half)], sw0)
        g1.wait()
        w1 = pltpu.async_copy(r1, out_hbm.at[pl.ds(base + half, half)], sw1)

        @pl.when(wid == 0)
        def _():
            pltpu.sync_copy(z_hbm, z_v)
            pltpu.sync_copy(z_v, out_hbm.at[pl.ds(B, 8)])

        w0.wait()
        w1.wait()

    return k(table, idx, zrows)


def _sc_dispatch_gather(h_z, cs_z, idx, B):
    bpw = B // _NW
    mesh = plsc.VectorSubcoreMesh(core_axis_name="c", subcore_axis_name="s")

    @functools.partial(
        pl.kernel, mesh=mesh,
        out_type=(
            jax.ShapeDtypeStruct((B, D_MODEL), jnp.float32),
            jax.ShapeDtypeStruct((B, 2 * HD), jnp.float32),
        ),
        scratch_types=[
            pltpu.VMEM((bpw,), jnp.int32),
            pltpu.VMEM((bpw // 2, D_MODEL), jnp.float32),
            pltpu.VMEM((bpw // 2, D_MODEL), jnp.float32),
            pltpu.VMEM((bpw, 2 * HD), jnp.float32),
            pltpu.SemaphoreType.DMA,
            pltpu.SemaphoreType.DMA,
            pltpu.SemaphoreType.DMA,
            pltpu.SemaphoreType.DMA,
            pltpu.SemaphoreType.DMA,
        ],
    )
    def k(h_hbm, cs_hbm, idx_hbm, xin_hbm, csr_hbm,
          idx_v, r0, r1, cs_v, sg0, sg1, sgc, sw0, sw1):
        wid = lax.axis_index("s") * _NC + lax.axis_index("c")
        base = wid * bpw
        half = bpw // 2
        pltpu.sync_copy(idx_hbm.at[pl.ds(base, bpw)], idx_v)
        g0 = pltpu.async_copy(h_hbm.at[idx_v.at[pl.ds(0, half)]], r0, sg0)
        g1 = pltpu.async_copy(h_hbm.at[idx_v.at[pl.ds(half, half)]], r1, sg1)
        gc = pltpu.async_copy(cs_hbm.at[idx_v], cs_v, sgc)
        g0.wait()
        w0 = pltpu.async_copy(r0, xin_hbm.at[pl.ds(base, half)], sw0)
        g1.wait()
        w1 = pltpu.async_copy(r1, xin_hbm.at[pl.ds(base + half, half)], sw1)
        gc.wait()
        pltpu.sync_copy(cs_v, csr_hbm.at[pl.ds(base, bpw)])
        w0.wait()
        w1.wait()

    return k(h_z, cs_z, idx)


def _sc_combine_gather(eo, pos1, pos2, B):
    bpw = B // _NW
    mesh = plsc.VectorSubcoreMesh(core_axis_name="c", subcore_axis_name="s")

    @functools.partial(
        pl.kernel, mesh=mesh,
        out_type=(
            jax.ShapeDtypeStruct((B, D_MODEL), jnp.float32),
            jax.ShapeDtypeStruct((B, D_MODEL), jnp.float32),
        ),
        scratch_types=[
            pltpu.VMEM((bpw,), jnp.int32),
            pltpu.VMEM((bpw,), jnp.int32),
            pltpu.VMEM((bpw // 2, D_MODEL), jnp.float32),
            pltpu.VMEM((bpw // 2, D_MODEL), jnp.float32),
            pltpu.SemaphoreType.DMA,
            pltpu.SemaphoreType.DMA,
            pltpu.SemaphoreType.DMA,
            pltpu.SemaphoreType.DMA,
        ],
    )
    def k(eo_hbm, p1_hbm, p2_hbm, g1_hbm, g2_hbm,
          i1, i2, r0, r1, sg0, sg1, sw0, sw1):
        wid = lax.axis_index("s") * _NC + lax.axis_index("c")
        base = wid * bpw
        half = bpw // 2
        pltpu.sync_copy(p1_hbm.at[pl.ds(base, bpw)], i1)
        pltpu.sync_copy(p2_hbm.at[pl.ds(base, bpw)], i2)
        ga = pltpu.async_copy(eo_hbm.at[i1.at[pl.ds(0, half)]], r0, sg0)
        gb = pltpu.async_copy(eo_hbm.at[i1.at[pl.ds(half, half)]], r1, sg1)
        ga.wait()
        wa = pltpu.async_copy(r0, g1_hbm.at[pl.ds(base, half)], sw0)
        gb.wait()
        wb = pltpu.async_copy(r1, g1_hbm.at[pl.ds(base + half, half)], sw1)
        wa.wait()
        gc = pltpu.async_copy(eo_hbm.at[i2.at[pl.ds(0, half)]], r0, sg0)
        wb.wait()
        gd = pltpu.async_copy(eo_hbm.at[i2.at[pl.ds(half, half)]], r1, sg1)
        gc.wait()
        wc = pltpu.async_copy(r0, g2_hbm.at[pl.ds(base, half)], sw0)
        gd.wait()
        wd = pltpu.async_copy(r1, g2_hbm.at[pl.ds(base + half, half)], sw1)
        wc.wait()
        wd.wait()

    return k(eo, pos1, pos2)


def _cumsum_lanes(x):
    n = x.shape[1]
    s = 1
    while s < n:
        pad = jnp.zeros((x.shape[0], s), x.dtype)
        x = x + jnp.concatenate([pad, x[:, :-s]], axis=1)
        s *= 2
    return x


def _router_body(h_ref, wr_ref, idx_ref, pos1_ref, pos2_ref, w1_ref, w2_ref):
    h = h_ref[:T_TOK, :]
    wr = wr_ref[...]
    logits = lax.dot_general(wr, h, (((1,), (1,)), ((), ())),
                             preferred_element_type=jnp.float32)
    neg_inf = jnp.float32(-jnp.inf)
    row_i = lax.broadcasted_iota(jnp.int32, (16, 1), 0)
    real = row_i < 9
    lm = jnp.where(real, logits, neg_inf)
    mx = jnp.max(lm, axis=0, keepdims=True)
    ex = jnp.where(real, jnp.exp(lm - mx), 0.0)
    probs = ex / jnp.sum(ex, axis=0, keepdims=True)

    masks = []
    for i in range(N_MODULES):
        li = lm[i:i + 1, :]
        jm = row_i < i
        better = real & ((lm > li) | ((lm == li) & jm))
        cnt = jnp.sum(better.astype(jnp.int32), axis=0, keepdims=True)
        masks.append((cnt < TOPK_E).astype(jnp.int32))
    mask = jnp.concatenate(masks, axis=0) > 0

    gates = jnp.where(mask, probs[:N_MODULES, :], neg_inf)
    b = lax.bitcast_convert_type(gates, jnp.int32)
    keys = jnp.where(b >= 0, b, b ^ jnp.int32(0x7FFFFFFF))

    def bs_body(_, lohi):
        lo, hi = lohi
        d = hi - lo
        mid = lo + lax.shift_right_logical(d, 1)
        cnt = jnp.sum((keys > mid).astype(jnp.int32), axis=1, keepdims=True)
        q = cnt < CAP
        return jnp.where(q, lo, mid + 1), jnp.where(q, mid, hi)

    lo0 = jnp.full((N_MODULES, 1), jnp.iinfo(jnp.int32).min, jnp.int32)
    hi0 = jnp.full((N_MODULES, 1), jnp.iinfo(jnp.int32).max, jnp.int32)
    thr, _ = lax.fori_loop(0, 32, bs_body, (lo0, hi0))

    above = keys > thr
    n1 = jnp.sum(above.astype(jnp.int32), axis=1, keepdims=True)
    tie = (keys == thr) & mask
    tie_i = tie.astype(jnp.int32)
    tie_rank = _cumsum_lanes(tie_i) - tie_i
    kept = mask & (above | (tie & (tie_rank < (CAP - n1))))
    kept_i = kept.astype(jnp.int32)
    p = _cumsum_lanes(kept_i) - kept_i

    c_iota = lax.broadcasted_iota(jnp.int32, (CAP, 1), 0)
    CH = 512
    cols = []
    for e in range(N_MODULES):
        acc = jnp.zeros((CAP, 1), jnp.float32)
        for ch in range(T_TOK // CH):
            pch = p[e:e + 1, ch * CH:(ch + 1) * CH]
            kch = kept[e:e + 1, ch * CH:(ch + 1) * CH]
            m = (kch & (pch == c_iota)).astype(jnp.float32)
            tcol = (lax.broadcasted_iota(jnp.int32, (CH, 1), 0)
                    + (ch * CH + 1)).astype(jnp.float32)
            acc = acc + lax.dot_general(m, tcol, (((1,), (0,)), ((), ())),
                                        preferred_element_type=jnp.float32)
        acc_i = acc.astype(jnp.int32)
        cols.append(jnp.where(acc_i > 0, acc_i - 1, T_TOK))
    idx_ref[...] = jnp.concatenate(cols, axis=1)

    e_iota = lax.broadcasted_iota(jnp.int32, (N_MODULES, 1), 0)
    w_et = jnp.where(kept, probs[:N_MODULES, :], 0.0)
    pos_et = e_iota * CAP + p
    ef = jnp.min(jnp.where(kept, e_iota, N_MODULES), axis=0, keepdims=True)
    sel1 = kept & (e_iota == ef)
    kept2 = kept & (e_iota > ef)
    es = jnp.min(jnp.where(kept2, e_iota, N_MODULES), axis=0, keepdims=True)
    sel2 = kept2 & (e_iota == es)
    t_row = lax.broadcasted_iota(jnp.int32, (1, T_TOK), 1)
    pos1 = jnp.sum(jnp.where(sel1, pos_et, 0), axis=0, keepdims=True)
    pos2 = jnp.sum(jnp.where(sel2, pos_et, 0), axis=0, keepdims=True)
    pos1_ref[...] = jnp.where(ef < N_MODULES, pos1, t_row)
    w1_ref[...] = jnp.sum(jnp.where(sel1, w_et, 0.0), axis=0, keepdims=True)
    pos2_ref[...] = jnp.where(es < N_MODULES, pos2, t_row)
    w2_ref[...] = jnp.sum(jnp.where(sel2, w_et, 0.0), axis=0, keepdims=True)


def _router(h, wr_p):
    return pl.pallas_call(
        _router_body,
        out_shape=(
            jax.ShapeDtypeStruct((CAP, N_MODULES), jnp.int32),
            jax.ShapeDtypeStruct((1, T_TOK), jnp.int32),
            jax.ShapeDtypeStruct((1, T_TOK), jnp.int32),
            jax.ShapeDtypeStruct((1, T_TOK), jnp.float32),
            jax.ShapeDtypeStruct((1, T_TOK), jnp.float32),
        ),
    )(h, wr_p)


def _bf(x):
    return x.astype(jnp.bfloat16)


def _att_body(xin_ref, cs_ref, wqkv_ref, wo_ref, out_ref):
    x = xin_ref[0]
    cs = cs_ref[0]
    cos = cs[:, :HD]
    sin = cs[:, HD:]
    qkv = lax.dot_general(_bf(x), _bf(wqkv_ref[0]), (((1,), (1,)), ((), ())),
                          preferred_element_type=jnp.float32)
    scale = 1.0 / math.sqrt(HD)
    outs = []
    for hh in range(N_HEADS):
        q = qkv[:, hh * HD:(hh + 1) * HD]
        k = qkv[:, D_MODEL + hh * HD:D_MODEL + (hh + 1) * HD]
        v = qkv[:, 2 * D_MODEL + hh * HD:2 * D_MODEL + (hh + 1) * HD]
        qr = jnp.concatenate([-q[:, HD // 2:], q[:, :HD // 2]], axis=1)
        kr = jnp.concatenate([-k[:, HD // 2:], k[:, :HD // 2]], axis=1)
        q = q * cos + qr * sin
        k = k * cos + kr * sin
        s = lax.dot_general(_bf(q), _bf(k), (((1,), (1,)), ((), ())),
                            preferred_element_type=jnp.float32) * scale
        m = jnp.max(s, axis=1, keepdims=True)
        e = jnp.exp(s - m)
        a = e / jnp.sum(e, axis=1, keepdims=True)
        outs.append(lax.dot_general(_bf(a), _bf(v), (((1,), (0,)), ((), ())),
                                    preferred_element_type=jnp.float32))
    o = jnp.concatenate(outs, axis=1)
    out_ref[0] = lax.dot_general(_bf(o), _bf(wo_ref[0]), (((1,), (1,)), ((), ())),
                                 preferred_element_type=jnp.float32)


def _experts_body(xin_ref, cs_ref, wqkv_ref, wo_ref, w1_ref, w2_ref, out_ref):
    s = pl.program_id(1)

    @pl.when(s == 0)
    def _():
        _att_body(xin_ref, cs_ref, wqkv_ref, wo_ref, out_ref)

    @pl.when(s > 0)
    def _():
        x = xin_ref[0]
        a = lax.dot_general(_bf(x), _bf(w1_ref[0]), (((1,), (1,)), ((), ())),
                            preferred_element_type=jnp.float32)
        a = jax.nn.gelu(a)
        o = lax.dot_general(_bf(a), _bf(w2_ref[0]), (((1,), (1,)), ((), ())),
                            preferred_element_type=jnp.float32)

        @pl.when(s == 1)
        def _():
            out_ref[0] = o

        @pl.when(s > 1)
        def _():
            out_ref[0] += o


def _experts(xin, cs_r, Wqkv, Wo, W1, W2):
    na = N_MODULES // 2
    d_ff = W1.shape[1]
    dffb = 1024
    nj = d_ff // dffb
    return pl.pallas_call(
        _experts_body,
        grid=(na, nj + 1),
        in_specs=[
            pl.BlockSpec((1, CAP, D_MODEL),
                         lambda a, s: (2 * a + jnp.minimum(s, 1), 0, 0)),
            pl.BlockSpec((1, CAP, 2 * HD), lambda a, s: (2 * a, 0, 0)),
            pl.BlockSpec((1, 3 * D_MODEL, D_MODEL), lambda a, s: (a, 0, 0)),
            pl.BlockSpec((1, D_MODEL, D_MODEL), lambda a, s: (a, 0, 0)),
            pl.BlockSpec((1, dffb, D_MODEL),
                         lambda a, s: (a, jnp.maximum(s - 1, 0), 0)),
            pl.BlockSpec((1, D_MODEL, dffb),
                         lambda a, s: (a, 0, jnp.maximum(s - 1, 0))),
        ],
        out_specs=pl.BlockSpec((1, CAP, D_MODEL),
                               lambda a, s: (2 * a + jnp.minimum(s, 1), 0, 0)),
        out_shape=jax.ShapeDtypeStruct((N_MODULES, CAP, D_MODEL), jnp.float32),
    )(xin, cs_r, Wqkv, Wo, W1, W2)


def _col(row, n):
    ii = lax.broadcasted_iota(jnp.int32, (n, n), 0)
    jj = lax.broadcasted_iota(jnp.int32, (n, n), 1)
    eye = (ii == jj).astype(jnp.float32)
    return lax.dot_general(eye, row, (((1,), (1,)), ((), ())),
                           preferred_element_type=jnp.float32)


def _final_body(h_ref, g1_ref, g2_ref, w1_ref, w2_ref, lnw_ref, out_ref):
    n = h_ref.shape[0]
    wc1 = _col(w1_ref[...], n)
    wc2 = _col(w2_ref[...], n)
    out = (h_ref[...] * (1.0 - wc1 - wc2)
           + wc1 * g1_ref[...] + wc2 * g2_ref[...])
    ms = jnp.mean(out * out, axis=1, keepdims=True)
    out_ref[...] = out * lax.rsqrt(ms + 1e-6) * lnw_ref[...]


def _final(h, g1, g2, w1, w2, lnw):
    tb = 256
    nb = T_TOK // tb
    return pl.pallas_call(
        _final_body,
        grid=(nb,),
        in_specs=[
            pl.BlockSpec((tb, D_MODEL), lambda b: (b, 0)),
            pl.BlockSpec((tb, D_MODEL), lambda b: (b, 0)),
            pl.BlockSpec((tb, D_MODEL), lambda b: (b, 0)),
            pl.BlockSpec((1, tb), lambda b: (0, b)),
            pl.BlockSpec((1, tb), lambda b: (0, b)),
            pl.BlockSpec((1, D_MODEL), lambda b: (0, 0)),
        ],
        out_specs=pl.BlockSpec((tb, D_MODEL), lambda b: (b, 0)),
        out_shape=jax.ShapeDtypeStruct((T_TOK, D_MODEL), jnp.float32),
    )(h, g1, g2, w1, w2, lnw)


def kernel(ids, embed_table, Wr, Wqkv, Wo, W1, W2, ln_w):
    ids_flat = ids.reshape(-1).astype(jnp.int32)

    zrows = jnp.zeros((8, D_MODEL), jnp.float32)
    h_z = _sc_embed_gather(embed_table, ids_flat, zrows, T_TOK, D_MODEL)

    pos = jnp.arange(T_TOK, dtype=jnp.float32)
    inv = 1.0 / (ROPE_BASE_C ** (jnp.arange(0, HD, 2, dtype=jnp.float32) / HD))
    freqs = pos[:, None] * inv[None, :]
    c_half = jnp.cos(freqs)
    s_half = jnp.sin(freqs)
    cs = jnp.concatenate([c_half, c_half, s_half, s_half], axis=1)
    cs_z = jnp.zeros((T_TOK + 8, 2 * HD), jnp.float32).at[:T_TOK].set(cs)

    wr_p = jnp.zeros((16, D_MODEL), jnp.float32).at[:9].set(Wr)
    idx_ce, pos1, pos2, w1, w2 = _router(h_z, wr_p)
    idx_flat = idx_ce.T.reshape(-1)

    xin_flat, csr_flat = _sc_dispatch_gather(h_z, cs_z, idx_flat, N_MODULES * CAP)
    xin = xin_flat.reshape(N_MODULES, CAP, D_MODEL)
    cs_r = csr_flat.reshape(N_MODULES, CAP, 2 * HD)

    eo_full = _experts(xin, cs_r, Wqkv, Wo, W1, W2)
    eo = eo_full.reshape(N_MODULES * CAP, D_MODEL)

    g1, g2 = _sc_combine_gather(eo, pos1.reshape(-1), pos2.reshape(-1), T_TOK)

    return _final(h_z, g1, g2, w1, w2, ln_w.reshape(1, D_MODEL))


# speedup vs baseline: 1.0697x; 1.0697x over previous
import functools
import math

import jax
import jax.numpy as jnp
from jax import lax
from jax.experimental import pallas as pl
from jax.experimental.pallas import tpu as pltpu
from jax.experimental.pallas import tpu_sc as plsc

D_MODEL = 1024
N_HEADS = 16
N_MODULES = 8
TOPK_E = 2
CAP = 256
T_TOK = 2048
HD = D_MODEL // N_HEADS
ROPE_BASE_C = 10000.0

_NC = 2
_NS = 16
_NW = _NC * _NS


def _sc_embed_gather(table, idx, zrows, B, D):
    bpw = B // _NW
    mesh = plsc.VectorSubcoreMesh(core_axis_name="c", subcore_axis_name="s")

    @functools.partial(
        pl.kernel, mesh=mesh,
        out_type=jax.ShapeDtypeStruct((B + 8, D), jnp.float32),
        scratch_types=[
            pltpu.VMEM((bpw // 2,), jnp.int32),
            pltpu.VMEM((bpw // 2,), jnp.int32),
            pltpu.VMEM((bpw // 2, D), jnp.float32),
            pltpu.VMEM((bpw // 2, D), jnp.float32),
            pltpu.VMEM((8, D), jnp.float32),
            pltpu.SemaphoreType.DMA,
            pltpu.SemaphoreType.DMA,
            pltpu.SemaphoreType.DMA,
            pltpu.SemaphoreType.DMA,
        ],
    )
    def k(table_hbm, idx_hbm, z_hbm, out_hbm, i0, i1, r0, r1, z_v,
          sg0, sg1, sw0, sw1):
        wid = lax.axis_index("s") * _NC + lax.axis_index("c")
        base = wid * bpw
        half = bpw // 2
        pltpu.sync_copy(idx_hbm.at[pl.ds(base, half)], i0)
        g0 = pltpu.async_copy(table_hbm.at[i0], r0, sg0)
        pltpu.sync_copy(idx_hbm.at[pl.ds(base + half, half)], i1)
        g1 = pltpu.async_copy(table_hbm.at[i1], r1, sg1)
        g0.wait()
        w0 = pltpu.async_copy(r0, out_hbm.at[pl.ds(base, half)], sw0)
        g1.wait()
        w1 = pltpu.async_copy(r1, out_hbm.at[pl.ds(base + half, half)], sw1)

        @pl.when(wid == 0)
        def _():
            pltpu.sync_copy(z_hbm, z_v)
            pltpu.sync_copy(z_v, out_hbm.at[pl.ds(B, 8)])

        w0.wait()
        w1.wait()

    return k(table, idx, zrows)


def _sc_dispatch_gather(h_z, cs_z, idx, B):
    bpw = B // _NW
    mesh = plsc.VectorSubcoreMesh(core_axis_name="c", subcore_axis_name="s")

    @functools.partial(
        pl.kernel, mesh=mesh,
        out_type=(
            jax.ShapeDtypeStruct((B, D_MODEL), jnp.float32),
            jax.ShapeDtypeStruct((B, 2 * HD), jnp.float32),
        ),
        scratch_types=[
            pltpu.VMEM((bpw,), jnp.int32),
            pltpu.VMEM((bpw // 2, D_MODEL), jnp.float32),
            pltpu.VMEM((bpw // 2, D_MODEL), jnp.float32),
            pltpu.VMEM((bpw, 2 * HD), jnp.float32),
            pltpu.SemaphoreType.DMA,
            pltpu.SemaphoreType.DMA,
            pltpu.SemaphoreType.DMA,
            pltpu.SemaphoreType.DMA,
            pltpu.SemaphoreType.DMA,
        ],
    )
    def k(h_hbm, cs_hbm, idx_hbm, xin_hbm, csr_hbm,
          idx_v, r0, r1, cs_v, sg0, sg1, sgc, sw0, sw1):
        wid = lax.axis_index("s") * _NC + lax.axis_index("c")
        base = wid * bpw
        half = bpw // 2
        pltpu.sync_copy(idx_hbm.at[pl.ds(base, bpw)], idx_v)
        g0 = pltpu.async_copy(h_hbm.at[idx_v.at[pl.ds(0, half)]], r0, sg0)
        g1 = pltpu.async_copy(h_hbm.at[idx_v.at[pl.ds(half, half)]], r1, sg1)
        gc = pltpu.async_copy(cs_hbm.at[idx_v], cs_v, sgc)
        g0.wait()
        w0 = pltpu.async_copy(r0, xin_hbm.at[pl.ds(base, half)], sw0)
        g1.wait()
        w1 = pltpu.async_copy(r1, xin_hbm.at[pl.ds(base + half, half)], sw1)
        gc.wait()
        pltpu.sync_copy(cs_v, csr_hbm.at[pl.ds(base, bpw)])
        w0.wait()
        w1.wait()

    return k(h_z, cs_z, idx)


def _sc_combine_gather(eo, pos1, pos2, B):
    bpw = B // _NW
    mesh = plsc.VectorSubcoreMesh(core_axis_name="c", subcore_axis_name="s")

    @functools.partial(
        pl.kernel, mesh=mesh,
        out_type=(
            jax.ShapeDtypeStruct((B, D_MODEL), jnp.float32),
            jax.ShapeDtypeStruct((B, D_MODEL), jnp.float32),
        ),
        scratch_types=[
            pltpu.VMEM((bpw,), jnp.int32),
            pltpu.VMEM((bpw,), jnp.int32),
            pltpu.VMEM((bpw // 2, D_MODEL), jnp.float32),
            pltpu.VMEM((bpw // 2, D_MODEL), jnp.float32),
            pltpu.SemaphoreType.DMA,
            pltpu.SemaphoreType.DMA,
            pltpu.SemaphoreType.DMA,
            pltpu.SemaphoreType.DMA,
        ],
    )
    def k(eo_hbm, p1_hbm, p2_hbm, g1_hbm, g2_hbm,
          i1, i2, r0, r1, sg0, sg1, sw0, sw1):
        wid = lax.axis_index("s") * _NC + lax.axis_index("c")
        base = wid * bpw
        half = bpw // 2
        pltpu.sync_copy(p1_hbm.at[pl.ds(base, bpw)], i1)
        pltpu.sync_copy(p2_hbm.at[pl.ds(base, bpw)], i2)
        ga = pltpu.async_copy(eo_hbm.at[i1.at[pl.ds(0, half)]], r0, sg0)
        gb = pltpu.async_copy(eo_hbm.at[i1.at[pl.ds(half, half)]], r1, sg1)
        ga.wait()
        wa = pltpu.async_copy(r0, g1_hbm.at[pl.ds(base, half)], sw0)
        gb.wait()
        wb = pltpu.async_copy(r1, g1_hbm.at[pl.ds(base + half, half)], sw1)
        wa.wait()
        gc = pltpu.async_copy(eo_hbm.at[i2.at[pl.ds(0, half)]], r0, sg0)
        wb.wait()
        gd = pltpu.async_copy(eo_hbm.at[i2.at[pl.ds(half, half)]], r1, sg1)
        gc.wait()
        wc = pltpu.async_copy(r0, g2_hbm.at[pl.ds(base, half)], sw0)
        gd.wait()
        wd = pltpu.async_copy(r1, g2_hbm.at[pl.ds(base + half, half)], sw1)
        wc.wait()
        wd.wait()

    return k(eo, pos1, pos2)


def _cumsum_lanes(x):
    n = x.shape[1]
    s = 1
    while s < n:
        pad = jnp.zeros((x.shape[0], s), x.dtype)
        x = x + jnp.concatenate([pad, x[:, :-s]], axis=1)
        s *= 2
    return x


def _router_body(h_ref, wr_ref, idx_ref, pos1_ref, pos2_ref, w1_ref, w2_ref):
    h = h_ref[:T_TOK, :]
    wr = wr_ref[...]
    logits = lax.dot_general(wr, h, (((1,), (1,)), ((), ())),
                             preferred_element_type=jnp.float32)
    neg_inf = jnp.float32(-jnp.inf)
    row_i = lax.broadcasted_iota(jnp.int32, (16, 1), 0)
    real = row_i < 9
    lm = jnp.where(real, logits, neg_inf)
    mx = jnp.max(lm, axis=0, keepdims=True)
    ex = jnp.where(real, jnp.exp(lm - mx), 0.0)
    probs = ex / jnp.sum(ex, axis=0, keepdims=True)

    masks = []
    for i in range(N_MODULES):
        li = lm[i:i + 1, :]
        jm = row_i < i
        better = real & ((lm > li) | ((lm == li) & jm))
        cnt = jnp.sum(better.astype(jnp.int32), axis=0, keepdims=True)
        masks.append((cnt < TOPK_E).astype(jnp.int32))
    mask = jnp.concatenate(masks, axis=0) > 0

    gates = jnp.where(mask, probs[:N_MODULES, :], neg_inf)
    b = lax.bitcast_convert_type(gates, jnp.int32)
    keys = jnp.where(b >= 0, b, b ^ jnp.int32(0x7FFFFFFF))

    def bs_body(_, lohi):
        lo, hi = lohi
        d = hi - lo
        mid = lo + lax.shift_right_logical(d, 1)
        cnt = jnp.sum((keys > mid).astype(jnp.int32), axis=1, keepdims=True)
        q = cnt < CAP
        return jnp.where(q, lo, mid + 1), jnp.where(q, mid, hi)

    lo0 = jnp.full((N_MODULES, 1), jnp.iinfo(jnp.int32).min, jnp.int32)
    hi0 = jnp.full((N_MODULES, 1), jnp.iinfo(jnp.int32).max, jnp.int32)
    thr, _ = lax.fori_loop(0, 32, bs_body, (lo0, hi0))

    above = keys > thr
    n1 = jnp.sum(above.astype(jnp.int32), axis=1, keepdims=True)
    tie = (keys == thr) & mask
    tie_i = tie.astype(jnp.int32)
    tie_rank = _cumsum_lanes(tie_i) - tie_i
    kept = mask & (above | (tie & (tie_rank < (CAP - n1))))
    kept_i = kept.astype(jnp.int32)
    p = _cumsum_lanes(kept_i) - kept_i

    c_iota = lax.broadcasted_iota(jnp.int32, (CAP, 1), 0)
    CH = 128
    cols = []
    for e in range(N_MODULES):
        acc = jnp.zeros((CAP, 1), jnp.int32)
        for ch in range(T_TOK // CH):
            pch = p[e:e + 1, ch * CH:(ch + 1) * CH]
            kch = kept[e:e + 1, ch * CH:(ch + 1) * CH]
            m = kch & (pch == c_iota)
            tch = lax.broadcasted_iota(jnp.int32, (1, CH), 1) + (ch * CH + 1)
            acc = acc + jnp.sum(jnp.where(m, tch, 0), axis=1, keepdims=True)
        cols.append(jnp.where(acc > 0, acc - 1, T_TOK))
    idx_ref[...] = jnp.concatenate(cols, axis=1)

    e_iota = lax.broadcasted_iota(jnp.int32, (N_MODULES, 1), 0)
    w_et = jnp.where(kept, probs[:N_MODULES, :], 0.0)
    pos_et = e_iota * CAP + p
    ef = jnp.min(jnp.where(kept, e_iota, N_MODULES), axis=0, keepdims=True)
    sel1 = kept & (e_iota == ef)
    kept2 = kept & (e_iota > ef)
    es = jnp.min(jnp.where(kept2, e_iota, N_MODULES), axis=0, keepdims=True)
    sel2 = kept2 & (e_iota == es)
    t_row = lax.broadcasted_iota(jnp.int32, (1, T_TOK), 1)
    pos1 = jnp.sum(jnp.where(sel1, pos_et, 0), axis=0, keepdims=True)
    pos2 = jnp.sum(jnp.where(sel2, pos_et, 0), axis=0, keepdims=True)
    pos1_ref[...] = jnp.where(ef < N_MODULES, pos1, t_row)
    w1_ref[...] = jnp.sum(jnp.where(sel1, w_et, 0.0), axis=0, keepdims=True)
    pos2_ref[...] = jnp.where(es < N_MODULES, pos2, t_row)
    w2_ref[...] = jnp.sum(jnp.where(sel2, w_et, 0.0), axis=0, keepdims=True)


def _router(h, wr_p):
    return pl.pallas_call(
        _router_body,
        out_shape=(
            jax.ShapeDtypeStruct((CAP, N_MODULES), jnp.int32),
            jax.ShapeDtypeStruct((1, T_TOK), jnp.int32),
            jax.ShapeDtypeStruct((1, T_TOK), jnp.int32),
            jax.ShapeDtypeStruct((1, T_TOK), jnp.float32),
            jax.ShapeDtypeStruct((1, T_TOK), jnp.float32),
        ),
    )(h, wr_p)


def _bf(x):
    return x.astype(jnp.bfloat16)


def _att_body(xin_ref, cs_ref, wqkv_ref, wo_ref, out_ref):
    x = xin_ref[0]
    cs = cs_ref[0]
    cos = cs[:, :HD]
    sin = cs[:, HD:]
    qkv = lax.dot_general(_bf(x), _bf(wqkv_ref[0]), (((1,), (1,)), ((), ())),
                          preferred_element_type=jnp.float32)
    scale = 1.0 / math.sqrt(HD)
    outs = []
    for hh in range(N_HEADS):
        q = qkv[:, hh * HD:(hh + 1) * HD]
        k = qkv[:, D_MODEL + hh * HD:D_MODEL + (hh + 1) * HD]
        v = qkv[:, 2 * D_MODEL + hh * HD:2 * D_MODEL + (hh + 1) * HD]
        qr = jnp.concatenate([-q[:, HD // 2:], q[:, :HD // 2]], axis=1)
        kr = jnp.concatenate([-k[:, HD // 2:], k[:, :HD // 2]], axis=1)
        q = q * cos + qr * sin
        k = k * cos + kr * sin
        s = lax.dot_general(_bf(q), _bf(k), (((1,), (1,)), ((), ())),
                            preferred_element_type=jnp.float32) * scale
        m = jnp.max(s, axis=1, keepdims=True)
        e = jnp.exp(s - m)
        a = e / jnp.sum(e, axis=1, keepdims=True)
        outs.append(lax.dot_general(_bf(a), _bf(v), (((1,), (0,)), ((), ())),
                                    preferred_element_type=jnp.float32))
    o = jnp.concatenate(outs, axis=1)
    out_ref[0] = lax.dot_general(_bf(o), _bf(wo_ref[0]), (((1,), (1,)), ((), ())),
                                 preferred_element_type=jnp.float32)


def _att_experts(xin, cs_r, Wqkv, Wo):
    na = N_MODULES // 2
    return pl.pallas_call(
        _att_body,
        grid=(na,),
        in_specs=[
            pl.BlockSpec((1, CAP, D_MODEL), lambda a: (2 * a, 0, 0)),
            pl.BlockSpec((1, CAP, 2 * HD), lambda a: (2 * a, 0, 0)),
            pl.BlockSpec((1, 3 * D_MODEL, D_MODEL), lambda a: (a, 0, 0)),
            pl.BlockSpec((1, D_MODEL, D_MODEL), lambda a: (a, 0, 0)),
        ],
        out_specs=pl.BlockSpec((1, CAP, D_MODEL), lambda a: (2 * a, 0, 0)),
        out_shape=jax.ShapeDtypeStruct((N_MODULES, CAP, D_MODEL), jnp.float32),
    )(xin, cs_r, Wqkv, Wo)


def _ffn_body(xin_ref, w1_ref, w2_ref, eo_any_ref, out_ref):
    del eo_any_ref
    j = pl.program_id(1)
    x = xin_ref[0]
    a = lax.dot_general(_bf(x), _bf(w1_ref[0]), (((1,), (1,)), ((), ())),
                        preferred_element_type=jnp.float32)
    a = jax.nn.gelu(a)
    o = lax.dot_general(_bf(a), _bf(w2_ref[0]), (((1,), (1,)), ((), ())),
                        preferred_element_type=jnp.float32)

    @pl.when(j == 0)
    def _():
        out_ref[0] = o

    @pl.when(j > 0)
    def _():
        out_ref[0] += o


def _ffn_experts(xin, W1, W2, eo_att):
    nf = N_MODULES // 2
    d_ff = W1.shape[1]
    dffb = 1024
    nj = d_ff // dffb
    return pl.pallas_call(
        _ffn_body,
        grid=(nf, nj),
        in_specs=[
            pl.BlockSpec((1, CAP, D_MODEL), lambda a, j: (2 * a + 1, 0, 0)),
            pl.BlockSpec((1, dffb, D_MODEL), lambda a, j: (a, j, 0)),
            pl.BlockSpec((1, D_MODEL, dffb), lambda a, j: (a, 0, j)),
            pl.BlockSpec(memory_space=pl.ANY),
        ],
        out_specs=pl.BlockSpec((1, CAP, D_MODEL), lambda a, j: (2 * a + 1, 0, 0)),
        out_shape=jax.ShapeDtypeStruct((N_MODULES, CAP, D_MODEL), jnp.float32),
        input_output_aliases={3: 0},
    )(xin, W1, W2, eo_att)


def _col(row, n):
    ii = lax.broadcasted_iota(jnp.int32, (n, n), 0)
    jj = lax.broadcasted_iota(jnp.int32, (n, n), 1)
    eye = (ii == jj).astype(jnp.float32)
    return lax.dot_general(eye, row, (((1,), (1,)), ((), ())),
                           preferred_element_type=jnp.float32)


def _final_body(h_ref, g1_ref, g2_ref, w1_ref, w2_ref, lnw_ref, out_ref):
    n = h_ref.shape[0]
    wc1 = _col(w1_ref[...], n)
    wc2 = _col(w2_ref[...], n)
    out = (h_ref[...] * (1.0 - wc1 - wc2)
           + wc1 * g1_ref[...] + wc2 * g2_ref[...])
    ms = jnp.mean(out * out, axis=1, keepdims=True)
    out_ref[...] = out * lax.rsqrt(ms + 1e-6) * lnw_ref[...]


def _final(h, g1, g2, w1, w2, lnw):
    tb = 256
    nb = T_TOK // tb
    return pl.pallas_call(
        _final_body,
        grid=(nb,),
        in_specs=[
            pl.BlockSpec((tb, D_MODEL), lambda b: (b, 0)),
            pl.BlockSpec((tb, D_MODEL), lambda b: (b, 0)),
            pl.BlockSpec((tb, D_MODEL), lambda b: (b, 0)),
            pl.BlockSpec((1, tb), lambda b: (0, b)),
            pl.BlockSpec((1, tb), lambda b: (0, b)),
            pl.BlockSpec((1, D_MODEL), lambda b: (0, 0)),
        ],
        out_specs=pl.BlockSpec((tb, D_MODEL), lambda b: (b, 0)),
        out_shape=jax.ShapeDtypeStruct((T_TOK, D_MODEL), jnp.float32),
    )(h, g1, g2, w1, w2, lnw)


def kernel(ids, embed_table, Wr, Wqkv, Wo, W1, W2, ln_w):
    ids_flat = ids.reshape(-1).astype(jnp.int32)

    zrows = jnp.zeros((8, D_MODEL), jnp.float32)
    h_z = _sc_embed_gather(embed_table, ids_flat, zrows, T_TOK, D_MODEL)

    pos = jnp.arange(T_TOK, dtype=jnp.float32)
    inv = 1.0 / (ROPE_BASE_C ** (jnp.arange(0, HD, 2, dtype=jnp.float32) / HD))
    freqs = pos[:, None] * inv[None, :]
    c_half = jnp.cos(freqs)
    s_half = jnp.sin(freqs)
    cs = jnp.concatenate([c_half, c_half, s_half, s_half], axis=1)
    cs_z = jnp.zeros((T_TOK + 8, 2 * HD), jnp.float32).at[:T_TOK].set(cs)

    wr_p = jnp.zeros((16, D_MODEL), jnp.float32).at[:9].set(Wr)
    idx_ce, pos1, pos2, w1, w2 = _router(h_z, wr_p)
    idx_flat = idx_ce.T.reshape(-1)

    xin_flat, csr_flat = _sc_dispatch_gather(h_z, cs_z, idx_flat, N_MODULES * CAP)
    xin = xin_flat.reshape(N_MODULES, CAP, D_MODEL)
    cs_r = csr_flat.reshape(N_MODULES, CAP, 2 * HD)

    eo_att = _att_experts(xin, cs_r, Wqkv, Wo)
    eo_full = _ffn_experts(xin, W1, W2, eo_att)
    eo = eo_full.reshape(N_MODULES * CAP, D_MODEL)

    g1, g2 = _sc_combine_gather(eo, pos1.reshape(-1), pos2.reshape(-1), T_TOK)

    return _final(h_z, g1, g2, w1, w2, ln_w.reshape(1, D_MODEL))


# speedup vs baseline: 1.1009x; 1.0291x over previous
import functools
import math

import jax
import jax.numpy as jnp
from jax import lax
from jax.experimental import pallas as pl
from jax.experimental.pallas import tpu as pltpu
from jax.experimental.pallas import tpu_sc as plsc

D_MODEL = 1024
N_HEADS = 16
N_MODULES = 8
TOPK_E = 2
CAP = 256
T_TOK = 2048
HD = D_MODEL // N_HEADS
ROPE_BASE_C = 10000.0

_NC = 2
_NS = 16
_NW = _NC * _NS


def _sc_embed_gather(table, idx, zrows, B, D):
    bpw = B // _NW
    mesh = plsc.VectorSubcoreMesh(core_axis_name="c", subcore_axis_name="s")

    @functools.partial(
        pl.kernel, mesh=mesh,
        out_type=jax.ShapeDtypeStruct((B + 8, D), jnp.float32),
        scratch_types=[
            pltpu.VMEM((bpw // 2,), jnp.int32),
            pltpu.VMEM((bpw // 2,), jnp.int32),
            pltpu.VMEM((bpw // 2, D), jnp.float32),
            pltpu.VMEM((bpw // 2, D), jnp.float32),
            pltpu.VMEM((8, D), jnp.float32),
            pltpu.SemaphoreType.DMA,
            pltpu.SemaphoreType.DMA,
            pltpu.SemaphoreType.DMA,
            pltpu.SemaphoreType.DMA,
        ],
    )
    def k(table_hbm, idx_hbm, z_hbm, out_hbm, i0, i1, r0, r1, z_v,
          sg0, sg1, sw0, sw1):
        wid = lax.axis_index("s") * _NC + lax.axis_index("c")
        base = wid * bpw
        half = bpw // 2
        pltpu.sync_copy(idx_hbm.at[pl.ds(base, half)], i0)
        g0 = pltpu.async_copy(table_hbm.at[i0], r0, sg0)
        pltpu.sync_copy(idx_hbm.at[pl.ds(base + half, half)], i1)
        g1 = pltpu.async_copy(table_hbm.at[i1], r1, sg1)
        g0.wait()
        w0 = pltpu.async_copy(r0, out_hbm.at[pl.ds(base, half)], sw0)
        g1.wait()
        w1 = pltpu.async_copy(r1, out_hbm.at[pl.ds(base + half, half)], sw1)

        @pl.when(wid == 0)
        def _():
            pltpu.sync_copy(z_hbm, z_v)
            pltpu.sync_copy(z_v, out_hbm.at[pl.ds(B, 8)])

        w0.wait()
        w1.wait()

    return k(table, idx, zrows)


def _sc_dispatch_gather(h_z, cs_z, idx, B):
    bpw = B // _NW
    mesh = plsc.VectorSubcoreMesh(core_axis_name="c", subcore_axis_name="s")

    @functools.partial(
        pl.kernel, mesh=mesh,
        out_type=(
            jax.ShapeDtypeStruct((B, D_MODEL), jnp.float32),
            jax.ShapeDtypeStruct((B, 2 * HD), jnp.float32),
        ),
        scratch_types=[
            pltpu.VMEM((bpw,), jnp.int32),
            pltpu.VMEM((bpw // 2, D_MODEL), jnp.float32),
            pltpu.VMEM((bpw // 2, D_MODEL), jnp.float32),
            pltpu.VMEM((bpw, 2 * HD), jnp.float32),
            pltpu.SemaphoreType.DMA,
            pltpu.SemaphoreType.DMA,
            pltpu.SemaphoreType.DMA,
            pltpu.SemaphoreType.DMA,
            pltpu.SemaphoreType.DMA,
        ],
    )
    def k(h_hbm, cs_hbm, idx_hbm, xin_hbm, csr_hbm,
          idx_v, r0, r1, cs_v, sg0, sg1, sgc, sw0, sw1):
        wid = lax.axis_index("s") * _NC + lax.axis_index("c")
        base = wid * bpw
        half = bpw // 2
        pltpu.sync_copy(idx_hbm.at[pl.ds(base, bpw)], idx_v)
        g0 = pltpu.async_copy(h_hbm.at[idx_v.at[pl.ds(0, half)]], r0, sg0)
        g1 = pltpu.async_copy(h_hbm.at[idx_v.at[pl.ds(half, half)]], r1, sg1)
        gc = pltpu.async_copy(cs_hbm.at[idx_v], cs_v, sgc)
        g0.wait()
        w0 = pltpu.async_copy(r0, xin_hbm.at[pl.ds(base, half)], sw0)
        g1.wait()
        w1 = pltpu.async_copy(r1, xin_hbm.at[pl.ds(base + half, half)], sw1)
        gc.wait()
        pltpu.sync_copy(cs_v, csr_hbm.at[pl.ds(base, bpw)])
        w0.wait()
        w1.wait()

    return k(h_z, cs_z, idx)


def _sc_combine_gather(eo, pos1, pos2, B):
    bpw = B // _NW
    mesh = plsc.VectorSubcoreMesh(core_axis_name="c", subcore_axis_name="s")

    @functools.partial(
        pl.kernel, mesh=mesh,
        out_type=(
            jax.ShapeDtypeStruct((B, D_MODEL), jnp.float32),
            jax.ShapeDtypeStruct((B, D_MODEL), jnp.float32),
        ),
        scratch_types=[
            pltpu.VMEM((bpw,), jnp.int32),
            pltpu.VMEM((bpw,), jnp.int32),
            pltpu.VMEM((bpw // 2, D_MODEL), jnp.float32),
            pltpu.VMEM((bpw // 2, D_MODEL), jnp.float32),
            pltpu.SemaphoreType.DMA,
            pltpu.SemaphoreType.DMA,
            pltpu.SemaphoreType.DMA,
            pltpu.SemaphoreType.DMA,
        ],
    )
    def k(eo_hbm, p1_hbm, p2_hbm, g1_hbm, g2_hbm,
          i1, i2, r0, r1, sg0, sg1, sw0, sw1):
        wid = lax.axis_index("s") * _NC + lax.axis_index("c")
        base = wid * bpw
        half = bpw // 2
        pltpu.sync_copy(p1_hbm.at[pl.ds(base, bpw)], i1)
        pltpu.sync_copy(p2_hbm.at[pl.ds(base, bpw)], i2)
        ga = pltpu.async_copy(eo_hbm.at[i1.at[pl.ds(0, half)]], r0, sg0)
        gb = pltpu.async_copy(eo_hbm.at[i1.at[pl.ds(half, half)]], r1, sg1)
        ga.wait()
        wa = pltpu.async_copy(r0, g1_hbm.at[pl.ds(base, half)], sw0)
        gb.wait()
        wb = pltpu.async_copy(r1, g1_hbm.at[pl.ds(base + half, half)], sw1)
        wa.wait()
        gc = pltpu.async_copy(eo_hbm.at[i2.at[pl.ds(0, half)]], r0, sg0)
        wb.wait()
        gd = pltpu.async_copy(eo_hbm.at[i2.at[pl.ds(half, half)]], r1, sg1)
        gc.wait()
        wc = pltpu.async_copy(r0, g2_hbm.at[pl.ds(base, half)], sw0)
        gd.wait()
        wd = pltpu.async_copy(r1, g2_hbm.at[pl.ds(base + half, half)], sw1)
        wc.wait()
        wd.wait()

    return k(eo, pos1, pos2)


def _cumsum_lanes(x):
    n = x.shape[1]
    s = 1
    while s < n:
        pad = jnp.zeros((x.shape[0], s), x.dtype)
        x = x + jnp.concatenate([pad, x[:, :-s]], axis=1)
        s *= 2
    return x


def _router_body(h_ref, wr_ref, idx_ref, pos1_ref, pos2_ref, w1_ref, w2_ref):
    h = h_ref[:T_TOK, :]
    wr = wr_ref[...]
    logits = lax.dot_general(wr, h, (((1,), (1,)), ((), ())),
                             preferred_element_type=jnp.float32)
    neg_inf = jnp.float32(-jnp.inf)
    row_i = lax.broadcasted_iota(jnp.int32, (16, 1), 0)
    real = row_i < 9
    lm = jnp.where(real, logits, neg_inf)
    mx = jnp.max(lm, axis=0, keepdims=True)
    ex = jnp.where(real, jnp.exp(lm - mx), 0.0)
    probs = ex / jnp.sum(ex, axis=0, keepdims=True)

    masks = []
    for i in range(N_MODULES):
        li = lm[i:i + 1, :]
        jm = row_i < i
        better = real & ((lm > li) | ((lm == li) & jm))
        cnt = jnp.sum(better.astype(jnp.int32), axis=0, keepdims=True)
        masks.append((cnt < TOPK_E).astype(jnp.int32))
    mask = jnp.concatenate(masks, axis=0) > 0

    gates = jnp.where(mask, probs[:N_MODULES, :], neg_inf)
    b = lax.bitcast_convert_type(gates, jnp.int32)
    keys = jnp.where(b >= 0, b, b ^ jnp.int32(0x7FFFFFFF))

    def bs_body(_, lohi):
        lo, hi = lohi
        d = hi - lo
        mid = lo + lax.shift_right_logical(d, 1)
        cnt = jnp.sum((keys > mid).astype(jnp.int32), axis=1, keepdims=True)
        q = cnt < CAP
        return jnp.where(q, lo, mid + 1), jnp.where(q, mid, hi)

    lo0 = jnp.full((N_MODULES, 1), jnp.iinfo(jnp.int32).min, jnp.int32)
    hi0 = jnp.full((N_MODULES, 1), jnp.iinfo(jnp.int32).max, jnp.int32)
    thr, _ = lax.fori_loop(0, 32, bs_body, (lo0, hi0))

    above = keys > thr
    n1 = jnp.sum(above.astype(jnp.int32), axis=1, keepdims=True)
    tie = (keys == thr) & mask
    tie_i = tie.astype(jnp.int32)
    tie_rank = _cumsum_lanes(tie_i) - tie_i
    kept = mask & (above | (tie & (tie_rank < (CAP - n1))))
    kept_i = kept.astype(jnp.int32)
    p = _cumsum_lanes(kept_i) - kept_i

    c_iota = lax.broadcasted_iota(jnp.int32, (CAP, 1), 0)
    CH = 512
    cols = []
    for e in range(N_MODULES):
        acc_hi = jnp.zeros((CAP, 1), jnp.float32)
        acc_lo = jnp.zeros((CAP, 1), jnp.float32)
        for ch in range(T_TOK // CH):
            pch = p[e:e + 1, ch * CH:(ch + 1) * CH]
            kch = kept[e:e + 1, ch * CH:(ch + 1) * CH]
            m = (kch & (pch == c_iota)).astype(jnp.bfloat16)
            tch = lax.broadcasted_iota(jnp.int32, (CH, 1), 0) + (ch * CH + 1)
            t_hi = (tch // 64).astype(jnp.bfloat16)
            t_lo = (tch % 64).astype(jnp.bfloat16)
            acc_hi = acc_hi + lax.dot_general(
                m, t_hi, (((1,), (0,)), ((), ())),
                preferred_element_type=jnp.float32)
            acc_lo = acc_lo + lax.dot_general(
                m, t_lo, (((1,), (0,)), ((), ())),
                preferred_element_type=jnp.float32)
        acc = acc_hi.astype(jnp.int32) * 64 + acc_lo.astype(jnp.int32)
        cols.append(jnp.where(acc > 0, acc - 1, T_TOK))
    idx_ref[...] = jnp.concatenate(cols, axis=1)

    e_iota = lax.broadcasted_iota(jnp.int32, (N_MODULES, 1), 0)
    w_et = jnp.where(kept, probs[:N_MODULES, :], 0.0)
    pos_et = e_iota * CAP + p
    ef = jnp.min(jnp.where(kept, e_iota, N_MODULES), axis=0, keepdims=True)
    sel1 = kept & (e_iota == ef)
    kept2 = kept & (e_iota > ef)
    es = jnp.min(jnp.where(kept2, e_iota, N_MODULES), axis=0, keepdims=True)
    sel2 = kept2 & (e_iota == es)
    t_row = lax.broadcasted_iota(jnp.int32, (1, T_TOK), 1)
    pos1 = jnp.sum(jnp.where(sel1, pos_et, 0), axis=0, keepdims=True)
    pos2 = jnp.sum(jnp.where(sel2, pos_et, 0), axis=0, keepdims=True)
    pos1_ref[...] = jnp.where(ef < N_MODULES, pos1, t_row)
    w1_ref[...] = jnp.sum(jnp.where(sel1, w_et, 0.0), axis=0, keepdims=True)
    pos2_ref[...] = jnp.where(es < N_MODULES, pos2, t_row)
    w2_ref[...] = jnp.sum(jnp.where(sel2, w_et, 0.0), axis=0, keepdims=True)


def _router(h, wr_p):
    return pl.pallas_call(
        _router_body,
        out_shape=(
            jax.ShapeDtypeStruct((CAP, N_MODULES), jnp.int32),
            jax.ShapeDtypeStruct((1, T_TOK), jnp.int32),
            jax.ShapeDtypeStruct((1, T_TOK), jnp.int32),
            jax.ShapeDtypeStruct((1, T_TOK), jnp.float32),
            jax.ShapeDtypeStruct((1, T_TOK), jnp.float32),
        ),
    )(h, wr_p)


def _bf(x):
    return x.astype(jnp.bfloat16)


def _att_body(xin_ref, cs_ref, wqkv_ref, wo_ref, out_ref):
    x = xin_ref[0]
    cs = cs_ref[0]
    cos = cs[:, :HD]
    sin = cs[:, HD:]
    qkv = lax.dot_general(_bf(x), _bf(wqkv_ref[0]), (((1,), (1,)), ((), ())),
                          preferred_element_type=jnp.float32)
    scale = 1.0 / math.sqrt(HD)
    outs = []
    for hh in range(N_HEADS):
        q = qkv[:, hh * HD:(hh + 1) * HD]
        k = qkv[:, D_MODEL + hh * HD:D_MODEL + (hh + 1) * HD]
        v = qkv[:, 2 * D_MODEL + hh * HD:2 * D_MODEL + (hh + 1) * HD]
        qr = jnp.concatenate([-q[:, HD // 2:], q[:, :HD // 2]], axis=1)
        kr = jnp.concatenate([-k[:, HD // 2:], k[:, :HD // 2]], axis=1)
        q = q * cos + qr * sin
        k = k * cos + kr * sin
        s = lax.dot_general(_bf(q), _bf(k), (((1,), (1,)), ((), ())),
                            preferred_element_type=jnp.float32) * scale
        m = jnp.max(s, axis=1, keepdims=True)
        e = jnp.exp(s - m)
        a = e / jnp.sum(e, axis=1, keepdims=True)
        outs.append(lax.dot_general(_bf(a), _bf(v), (((1,), (0,)), ((), ())),
                                    preferred_element_type=jnp.float32))
    o = jnp.concatenate(outs, axis=1)
    out_ref[0] = lax.dot_general(_bf(o), _bf(wo_ref[0]), (((1,), (1,)), ((), ())),
                                 preferred_element_type=jnp.float32)


def _att_experts(xin, cs_r, Wqkv, Wo):
    na = N_MODULES // 2
    return pl.pallas_call(
        _att_body,
        grid=(na,),
        in_specs=[
            pl.BlockSpec((1, CAP, D_MODEL), lambda a: (2 * a, 0, 0)),
            pl.BlockSpec((1, CAP, 2 * HD), lambda a: (2 * a, 0, 0)),
            pl.BlockSpec((1, 3 * D_MODEL, D_MODEL), lambda a: (a, 0, 0)),
            pl.BlockSpec((1, D_MODEL, D_MODEL), lambda a: (a, 0, 0)),
        ],
        out_specs=pl.BlockSpec((1, CAP, D_MODEL), lambda a: (2 * a, 0, 0)),
        out_shape=jax.ShapeDtypeStruct((N_MODULES, CAP, D_MODEL), jnp.float32),
    )(xin, cs_r, Wqkv, Wo)


def _ffn_body(xin_ref, w1_ref, w2_ref, eo_any_ref, out_ref):
    del eo_any_ref
    j = pl.program_id(1)
    x = xin_ref[0]
    a = lax.dot_general(_bf(x), _bf(w1_ref[0]), (((1,), (1,)), ((), ())),
                        preferred_element_type=jnp.float32)
    a = jax.nn.gelu(a)
    o = lax.dot_general(_bf(a), _bf(w2_ref[0]), (((1,), (1,)), ((), ())),
                        preferred_element_type=jnp.float32)

    @pl.when(j == 0)
    def _():
        out_ref[0] = o

    @pl.when(j > 0)
    def _():
        out_ref[0] += o


def _ffn_experts(xin, W1, W2, eo_att):
    nf = N_MODULES // 2
    d_ff = W1.shape[1]
    dffb = 2048
    nj = d_ff // dffb
    return pl.pallas_call(
        _ffn_body,
        grid=(nf, nj),
        in_specs=[
            pl.BlockSpec((1, CAP, D_MODEL), lambda a, j: (2 * a + 1, 0, 0)),
            pl.BlockSpec((1, dffb, D_MODEL), lambda a, j: (a, j, 0)),
            pl.BlockSpec((1, D_MODEL, dffb), lambda a, j: (a, 0, j)),
            pl.BlockSpec(memory_space=pl.ANY),
        ],
        out_specs=pl.BlockSpec((1, CAP, D_MODEL), lambda a, j: (2 * a + 1, 0, 0)),
        out_shape=jax.ShapeDtypeStruct((N_MODULES, CAP, D_MODEL), jnp.float32),
        input_output_aliases={3: 0},
    )(xin, W1, W2, eo_att)


def _col(row, n):
    ii = lax.broadcasted_iota(jnp.int32, (n, n), 0)
    jj = lax.broadcasted_iota(jnp.int32, (n, n), 1)
    eye = (ii == jj).astype(jnp.float32)
    return lax.dot_general(eye, row, (((1,), (1,)), ((), ())),
                           preferred_element_type=jnp.float32)


def _final_body(h_ref, g1_ref, g2_ref, w1_ref, w2_ref, lnw_ref, out_ref):
    n = h_ref.shape[0]
    wc1 = _col(w1_ref[...], n)
    wc2 = _col(w2_ref[...], n)
    out = (h_ref[...] * (1.0 - wc1 - wc2)
           + wc1 * g1_ref[...] + wc2 * g2_ref[...])
    ms = jnp.mean(out * out, axis=1, keepdims=True)
    out_ref[...] = out * lax.rsqrt(ms + 1e-6) * lnw_ref[...]


def _final(h, g1, g2, w1, w2, lnw):
    tb = 256
    nb = T_TOK // tb
    return pl.pallas_call(
        _final_body,
        grid=(nb,),
        in_specs=[
            pl.BlockSpec((tb, D_MODEL), lambda b: (b, 0)),
            pl.BlockSpec((tb, D_MODEL), lambda b: (b, 0)),
            pl.BlockSpec((tb, D_MODEL), lambda b: (b, 0)),
            pl.BlockSpec((1, tb), lambda b: (0, b)),
            pl.BlockSpec((1, tb), lambda b: (0, b)),
            pl.BlockSpec((1, D_MODEL), lambda b: (0, 0)),
        ],
        out_specs=pl.BlockSpec((tb, D_MODEL), lambda b: (b, 0)),
        out_shape=jax.ShapeDtypeStruct((T_TOK, D_MODEL), jnp.float32),
    )(h, g1, g2, w1, w2, lnw)


def kernel(ids, embed_table, Wr, Wqkv, Wo, W1, W2, ln_w):
    ids_flat = ids.reshape(-1).astype(jnp.int32)

    zrows = jnp.zeros((8, D_MODEL), jnp.float32)
    h_z = _sc_embed_gather(embed_table, ids_flat, zrows, T_TOK, D_MODEL)

    pos = jnp.arange(T_TOK, dtype=jnp.float32)
    inv = 1.0 / (ROPE_BASE_C ** (jnp.arange(0, HD, 2, dtype=jnp.float32) / HD))
    freqs = pos[:, None] * inv[None, :]
    c_half = jnp.cos(freqs)
    s_half = jnp.sin(freqs)
    cs = jnp.concatenate([c_half, c_half, s_half, s_half], axis=1)
    cs_z = jnp.zeros((T_TOK + 8, 2 * HD), jnp.float32).at[:T_TOK].set(cs)

    wr_p = jnp.zeros((16, D_MODEL), jnp.float32).at[:9].set(Wr)
    idx_ce, pos1, pos2, w1, w2 = _router(h_z, wr_p)
    idx_flat = idx_ce.T.reshape(-1)

    xin_flat, csr_flat = _sc_dispatch_gather(h_z, cs_z, idx_flat, N_MODULES * CAP)
    xin = xin_flat.reshape(N_MODULES, CAP, D_MODEL)
    cs_r = csr_flat.reshape(N_MODULES, CAP, 2 * HD)

    eo_att = _att_experts(xin, cs_r, Wqkv, Wo)
    eo_full = _ffn_experts(xin, W1, W2, eo_att)
    eo = eo_full.reshape(N_MODULES * CAP, D_MODEL)

    g1, g2 = _sc_combine_gather(eo, pos1.reshape(-1), pos2.reshape(-1), T_TOK)

    return _final(h_z, g1, g2, w1, w2, ln_w.reshape(1, D_MODEL))
